# baseline (speedup 1.0000x reference)
import jax
import jax.numpy as jnp
from jax.experimental import pallas as pl
from jax.experimental.pallas import tpu as pltpu

_BO = 512


def _fused_kernel(perm_ref, w_ref, bias_ref, x_ref, out_ref,
                  xbf_ref, wd_ref):
    j = pl.program_id(0)
    k = x_ref.shape[1]
    bo = out_ref.shape[1]
    cur = jax.lax.rem(j, 2)

    @pl.when(j == 0)
    def _cast_x():
        xbf_ref[...] = x_ref[...].astype(jnp.bfloat16)

    perm = perm_ref[...]
    wv = w_ref[...].astype(jnp.bfloat16)
    row = jax.lax.broadcasted_iota(jnp.int16, (k, bo), 0)
    acc = jnp.zeros((k, bo), jnp.bfloat16)
    for s in range(perm.shape[0]):
        acc = acc + jnp.where(
            row == perm[s : s + 1, :], wv[s : s + 1, :], jnp.bfloat16(0.0)
        )
    wd_ref[cur] = acc

    out_ref[...] = (
        jnp.dot(
            xbf_ref[...], wd_ref[1 - cur], preferred_element_type=jnp.float32
        )
        + bias_ref[...]
    )


def kernel(x, permutations, weight, bias):
    lead = x.shape[:-1]
    k = x.shape[-1]
    t = 1
    for d in lead:
        t *= d
    x2 = x.reshape(t, k)
    o, s = permutations.shape
    perm_t = permutations.T.astype(jnp.int16)
    w_t = weight.T
    bias2 = bias.reshape(1, o)
    nj = o // _BO
    out = pl.pallas_call(
        _fused_kernel,
        grid=(nj + 1,),
        in_specs=[
            pl.BlockSpec((s, _BO), lambda j: (0, jnp.minimum(j, nj - 1))),
            pl.BlockSpec((s, _BO), lambda j: (0, jnp.minimum(j, nj - 1))),
            pl.BlockSpec((1, _BO), lambda j: (0, jnp.maximum(j - 1, 0))),
            pl.BlockSpec((t, k), lambda j: (0, 0)),
        ],
        out_specs=pl.BlockSpec((t, _BO), lambda j: (0, jnp.maximum(j - 1, 0))),
        out_shape=jax.ShapeDtypeStruct((t, o), jnp.float32),
        scratch_shapes=[
            pltpu.VMEM((t, k), jnp.bfloat16),
            pltpu.VMEM((2, k, _BO), jnp.bfloat16),
        ],
    )(perm_t, w_t, bias2, x2)
    return out.reshape(*lead, o)


# speedup vs baseline: 1.1929x; 1.1929x over previous
import jax
import jax.numpy as jnp
from jax.experimental import pallas as pl
from jax.experimental.pallas import tpu as pltpu

_BO = 512


_RC = 64


def _fused_kernel(perm_ref, w_ref, bias_ref, x_ref, out_ref, xbf_ref, wd_ref):
    k = x_ref.shape[1]
    bo = out_ref.shape[1]

    @pl.when(pl.program_id(0) == 0)
    def _cast_x():
        xbf_ref[...] = x_ref[...].astype(jnp.bfloat16)

    perm = perm_ref[...]
    wv = w_ref[...].astype(jnp.bfloat16)
    base = jax.lax.broadcasted_iota(jnp.int16, (_RC, bo), 0)
    for c in range(k // _RC):
        row = base + jnp.int16(c * _RC)
        acc = jnp.where(row == perm[0:1, :], wv[0:1, :], jnp.bfloat16(0.0))
        for s in range(1, perm.shape[0]):
            acc = acc + jnp.where(
                row == perm[s : s + 1, :], wv[s : s + 1, :], jnp.bfloat16(0.0)
            )
        wd_ref[pl.ds(c * _RC, _RC), :] = acc
    out_ref[...] = (
        jnp.dot(xbf_ref[...], wd_ref[...], preferred_element_type=jnp.float32)
        + bias_ref[...]
    )


def kernel(x, permutations, weight, bias):
    lead = x.shape[:-1]
    k = x.shape[-1]
    t = 1
    for d in lead:
        t *= d
    x2 = x.reshape(t, k)
    o, s = permutations.shape
    perm_t = permutations.T.astype(jnp.int16)
    w_t = weight.T
    bias2 = bias.reshape(1, o)
    nj = o // _BO
    out = pl.pallas_call(
        _fused_kernel,
        grid=(nj,),
        in_specs=[
            pl.BlockSpec((s, _BO), lambda j: (0, j)),
            pl.BlockSpec((s, _BO), lambda j: (0, j)),
            pl.BlockSpec((1, _BO), lambda j: (0, j)),
            pl.BlockSpec((t, k), lambda j: (0, 0)),
        ],
        out_specs=pl.BlockSpec((t, _BO), lambda j: (0, j)),
        out_shape=jax.ShapeDtypeStruct((t, o), jnp.float32),
        scratch_shapes=[
            pltpu.VMEM((t, k), jnp.bfloat16),
            pltpu.VMEM((k, _BO), jnp.bfloat16),
        ],
    )(perm_t, w_t, bias2, x2)
    return out.reshape(*lead, o)
